# baseline (speedup 1.0000x reference)
import functools

import jax
import jax.numpy as jnp
from jax import lax
from jax.experimental import pallas as pl
from jax.experimental.pallas import tpu as pltpu
from jax.experimental.pallas import tpu_sc as plsc

VOCAB = 1000
VOCAB_PAD = 1024
BATCH = 4096
SEQ = 50
SEQ_PAD = 56
NC, NS = 2, 16
NW = NC * NS
B_PER_W = BATCH // NW
MAIN = 896
TAIL = VOCAB - MAIN
ROWS8 = 48
TAIL_OFFS = (0, 16, 32, 48, 64, 80, 88)


def _sc_gather(table_pad, idx3):
  mesh = plsc.VectorSubcoreMesh(core_axis_name="c", subcore_axis_name="s",
                                num_cores=NC, num_subcores=NS)

  @functools.partial(
      pl.kernel,
      out_type=jax.ShapeDtypeStruct((BATCH, SEQ, VOCAB), jnp.float32),
      mesh=mesh,
      scratch_types=[
          pltpu.VMEM((1, SEQ_PAD), jnp.int32),
          pltpu.VMEM((SEQ_PAD, VOCAB_PAD), jnp.float32),
          pltpu.VMEM((SEQ_PAD, VOCAB_PAD), jnp.float32),
          pltpu.VMEM((2, VOCAB), jnp.float32),
          pltpu.VMEM((ROWS8, TAIL), jnp.float32),
          pltpu.SemaphoreType.DMA,
          pltpu.SemaphoreType.DMA,
          pltpu.SemaphoreType.DMA,
          pltpu.SemaphoreType.DMA,
          pltpu.SemaphoreType.DMA,
          pltpu.SemaphoreType.DMA,
          pltpu.SemaphoreType.DMA,
      ],
  )
  def k(table_hbm, idx_hbm, out_hbm, idxb, pad0, pad1, tail2, tailc,
        isem, g0, g1, wm0, wm1, wt2, wtc):
    wid = lax.axis_index("s") * NC + lax.axis_index("c")
    base = wid * B_PER_W
    pads = (pad0, pad1)
    gsem = (g0, g1)
    wmsem = (wm0, wm1)

    pltpu.sync_copy(idx_hbm.at[base], idxb)
    pltpu.async_copy(table_hbm.at[idxb.at[0]], pad0, g0)

    def body(j, _):
      for s in range(2):
        jj = j + s
        ns = 1 - s
        bb = base + jj
        pltpu.make_async_copy(table_hbm.at[idxb.at[0]], pads[s],
                              gsem[s]).wait()

        @pl.when(jj + 1 < B_PER_W)
        def _():
          pltpu.async_copy(idx_hbm.at[bb + 1], idxb, isem)

        pltpu.async_copy(
            pads[s].at[pl.ds(0, ROWS8), pl.ds(0, MAIN)],
            out_hbm.at[bb, pl.ds(0, ROWS8), pl.ds(0, MAIN)], wmsem[s])

        @pl.when(jj >= 1)
        def _():
          pltpu.make_async_copy(
              tail2, out_hbm.at[bb - 1, pl.ds(ROWS8, 2)], wt2).wait()

        for r in range(2):
          for c in range(0, MAIN, 16):
            tail2[r, pl.ds(c, 16)] = pads[s][ROWS8 + r, pl.ds(c, 16)]
          for c in TAIL_OFFS:
            tail2[r, pl.ds(MAIN + c, 16)] = pads[s][ROWS8 + r,
                                                    pl.ds(MAIN + c, 16)]
        pltpu.async_copy(tail2, out_hbm.at[bb, pl.ds(ROWS8, 2)], wt2)

        @pl.when(jj >= 1)
        def _():
          pltpu.make_async_copy(
              tailc, out_hbm.at[bb - 1, pl.ds(0, ROWS8), pl.ds(MAIN, TAIL)],
              wtc).wait()

        for r in range(ROWS8):
          for c in TAIL_OFFS:
            tailc[r, pl.ds(c, 16)] = pads[s][r, pl.ds(MAIN + c, 16)]
        pltpu.async_copy(
            tailc, out_hbm.at[bb, pl.ds(0, ROWS8), pl.ds(MAIN, TAIL)], wtc)

        @pl.when(jj + 1 < B_PER_W)
        def _():
          pltpu.make_async_copy(idx_hbm.at[bb + 1], idxb, isem).wait()

          @pl.when(jj >= 1)
          def _():
            pltpu.make_async_copy(
                pads[ns].at[pl.ds(0, ROWS8), pl.ds(0, MAIN)],
                out_hbm.at[bb - 1, pl.ds(0, ROWS8), pl.ds(0, MAIN)],
                wmsem[ns]).wait()

          pltpu.async_copy(table_hbm.at[idxb.at[0]], pads[ns], gsem[ns])

      return ()

    lax.fori_loop(0, B_PER_W // 2, lambda i, c: body(i * 2, c), (),
                  unroll=False)
    last = base + B_PER_W - 1
    pltpu.make_async_copy(
        pad1.at[pl.ds(0, ROWS8), pl.ds(0, MAIN)],
        out_hbm.at[last, pl.ds(0, ROWS8), pl.ds(0, MAIN)], wm1).wait()
    pltpu.make_async_copy(tail2, out_hbm.at[last, pl.ds(ROWS8, 2)],
                          wt2).wait()
    pltpu.make_async_copy(
        tailc, out_hbm.at[last, pl.ds(0, ROWS8), pl.ds(MAIN, TAIL)],
        wtc).wait()

  return k(table_pad, idx3)


def kernel(idx, embedding_table):
  table_pad = jnp.pad(embedding_table, ((0, 0), (0, VOCAB_PAD - VOCAB)))
  idx_pad = jnp.pad(idx.astype(jnp.int32), ((0, 0), (0, SEQ_PAD - SEQ)))
  idx3 = idx_pad.reshape(BATCH, 1, SEQ_PAD)
  return _sc_gather(table_pad, idx3)
